# baseline (speedup 1.0000x reference)
import functools

import jax
import jax.numpy as jnp
from jax import lax
from jax.experimental import pallas as pl
from jax.experimental.pallas import tpu as pltpu
from jax.experimental.pallas import tpu_sc as plsc

_B = 4096
_S = 200
_D = 64
_V = 1000000

_NC = 2
_NS = 16
_NW = _NC * _NS
_C = _B // _NW

_mesh = plsc.VectorSubcoreMesh(core_axis_name="c", subcore_axis_name="s")


@functools.partial(
    pl.kernel,
    out_type=jax.ShapeDtypeStruct((_S, _D // 8, _B // _C, 8, _C), jnp.float32),
    mesh=_mesh,
    compiler_params=pltpu.CompilerParams(
        needs_layout_passes=False, use_tc_tiling_on_sc=False),
    scratch_types=[
        pltpu.VMEM((_C,), jnp.int32),
        pltpu.VMEM((_C,), jnp.int32),
        pltpu.VMEM((_C,), jnp.int32),
        pltpu.VMEM((_C, _D // 2), jnp.float32),
        pltpu.VMEM((_C, _D // 2), jnp.float32),
        pltpu.VMEM((_D, _C), jnp.float32),
        pltpu.VMEM((_S, _D), jnp.float32),
        pltpu.SemaphoreType.DMA,
    ],
)
def _embed(ids_hbm, tok_hbm, pos_hbm, out_hbm,
           idx_v, eidx_v, oidx_v, even_v, odd_v, stage_v, pos_v, sem):
    wid = lax.axis_index("s") * _NC + lax.axis_index("c")
    pltpu.sync_copy(pos_hbm, pos_v)
    lane = jnp.arange(16, dtype=jnp.int32)

    def chunk_body(s, carry):
        sh = lax.div(s, 8)
        sl = lax.rem(s, 8)
        pltpu.sync_copy(ids_hbm.at[sh, wid, sl], idx_v)

        @plsc.parallel_loop(0, _C // 16)
        def _expand(t):
            tv = idx_v[pl.ds(t * 16, 16)]
            tv2 = tv + tv
            eidx_v[pl.ds(t * 16, 16)] = tv2
            oidx_v[pl.ds(t * 16, 16)] = tv2 + 1

        cp_e = pltpu.async_copy(tok_hbm.at[eidx_v], even_v, sem)
        cp_o = pltpu.async_copy(tok_hbm.at[oidx_v], odd_v, sem)
        cp_e.wait()
        cp_o.wait()

        m1 = []
        rows16 = []
        for t in range(_C // 16):
            tv = idx_v[pl.ds(t * 16, 16)]
            m1.append(jnp.where(tv != 0, 1.0, 0.0))
            rows16.append(lane + (t * 16))

        @plsc.parallel_loop(0, _D // 2)
        def _col_e(d):
            p = plsc.load_gather(pos_v, [jnp.full((16,), s, jnp.int32),
                                         jnp.full((16,), d, jnp.int32)])
            col = jnp.full((16,), d, jnp.int32)
            for t in range(_C // 16):
                v = plsc.load_gather(even_v, [rows16[t], col])
                stage_v[d, pl.ds(t * 16, 16)] = (v * 8.0 + p) * m1[t]

        @plsc.parallel_loop(_D // 2, _D)
        def _col_o(d):
            p = plsc.load_gather(pos_v, [jnp.full((16,), s, jnp.int32),
                                         jnp.full((16,), d, jnp.int32)])
            col = jnp.full((16,), d - (_D // 2), jnp.int32)
            for t in range(_C // 16):
                v = plsc.load_gather(odd_v, [rows16[t], col])
                stage_v[d, pl.ds(t * 16, 16)] = (v * 8.0 + p) * m1[t]

        for dh in range(_D // 8):
            pltpu.sync_copy(stage_v.at[pl.ds(dh * 8, 8)], out_hbm.at[s, dh, wid])
        return carry

    lax.fori_loop(0, _S, chunk_body, 0)


def kernel(inputs, token_table, pos_table):
    ids4 = (inputs.T.reshape(_S // 8, 8, _B // _C, _C)
            .transpose(0, 2, 1, 3))
    tok_half = token_table.reshape(2 * _V, _D // 2)
    out5 = _embed(ids4, tok_half, pos_table)
    return (out5.transpose(2, 4, 0, 1, 3)
            .reshape(_B, _S, _D))


# speedup vs baseline: 1.2362x; 1.2362x over previous
import functools

import jax
import jax.numpy as jnp
from jax import lax
from jax.experimental import pallas as pl
from jax.experimental.pallas import tpu as pltpu
from jax.experimental.pallas import tpu_sc as plsc

_B = 4096
_S = 200
_D = 64
_V = 1000000

_NC = 2
_NS = 16
_NW = _NC * _NS
_C = _B // _NW
_H = _D // 2

_mesh = plsc.VectorSubcoreMesh(core_axis_name="c", subcore_axis_name="s")


@functools.partial(
    pl.kernel,
    out_type=jax.ShapeDtypeStruct((_S, _D // 8, _B // _C, 8, _C), jnp.float32),
    mesh=_mesh,
    compiler_params=pltpu.CompilerParams(
        needs_layout_passes=False, use_tc_tiling_on_sc=False),
    scratch_types=[
        pltpu.VMEM((2, _C), jnp.int32),
        pltpu.VMEM((2, _C), jnp.int32),
        pltpu.VMEM((2, _C), jnp.int32),
        pltpu.VMEM((2, _C, _H), jnp.float32),
        pltpu.VMEM((2, _C, _H), jnp.float32),
        pltpu.VMEM((2, _D, _C), jnp.float32),
        pltpu.VMEM((_S, _D), jnp.float32),
        pltpu.SemaphoreType.DMA((2,)),
        pltpu.SemaphoreType.DMA((2,)),
    ],
)
def _embed(ids_hbm, tok_hbm, pos_hbm, out_hbm,
           idx_v, eidx_v, oidx_v, even_v, odd_v, stage_v, pos_v,
           gsem, osem):
    wid = lax.axis_index("s") * _NC + lax.axis_index("c")
    pltpu.sync_copy(pos_hbm, pos_v)
    lane = jnp.arange(16, dtype=jnp.int32)

    def load_and_fire(s, j):
        sh = lax.div(s, 8)
        sl = lax.rem(s, 8)
        pltpu.sync_copy(ids_hbm.at[sh, wid, sl], idx_v.at[j])

        @plsc.parallel_loop(0, _C // 16)
        def _expand(t):
            tv = idx_v[j, pl.ds(t * 16, 16)]
            tv2 = tv + tv
            eidx_v[j, pl.ds(t * 16, 16)] = tv2
            oidx_v[j, pl.ds(t * 16, 16)] = tv2 + 1

        pltpu.async_copy(tok_hbm.at[eidx_v.at[j]], even_v.at[j], gsem.at[j])
        pltpu.async_copy(tok_hbm.at[oidx_v.at[j]], odd_v.at[j], gsem.at[j])

    def wait_gathers(j):
        pltpu.make_async_copy(tok_hbm.at[eidx_v.at[j]], even_v.at[j],
                              gsem.at[j]).wait()
        pltpu.make_async_copy(tok_hbm.at[oidx_v.at[j]], odd_v.at[j],
                              gsem.at[j]).wait()

    def fire_out(s, j):
        for dh in range(_D // 8):
            pltpu.async_copy(stage_v.at[j, pl.ds(dh * 8, 8)],
                             out_hbm.at[s, dh, wid], osem.at[j])

    def wait_out(s, j):
        for dh in range(_D // 8):
            pltpu.make_async_copy(stage_v.at[j, pl.ds(dh * 8, 8)],
                                  out_hbm.at[s, dh, wid], osem.at[j]).wait()

    def compute(s, j):
        wait_gathers(j)
        m1 = []
        rows16 = []
        for t in range(_C // 16):
            tv = idx_v[j, pl.ds(t * 16, 16)]
            m1.append(jnp.where(tv != 0, 1.0, 0.0))
            rows16.append(lane + (t * 16))

        @plsc.parallel_loop(0, _H)
        def _col_e(d):
            p = plsc.load_gather(pos_v, [jnp.full((16,), s, jnp.int32),
                                         jnp.full((16,), d, jnp.int32)])
            col = jnp.full((16,), d, jnp.int32)
            for t in range(_C // 16):
                v = plsc.load_gather(even_v, [jnp.full((16,), j, jnp.int32),
                                              rows16[t], col])
                stage_v[j, d, pl.ds(t * 16, 16)] = (v * 8.0 + p) * m1[t]

        @plsc.parallel_loop(_H, _D)
        def _col_o(d):
            p = plsc.load_gather(pos_v, [jnp.full((16,), s, jnp.int32),
                                         jnp.full((16,), d, jnp.int32)])
            col = jnp.full((16,), d - _H, jnp.int32)
            for t in range(_C // 16):
                v = plsc.load_gather(odd_v, [jnp.full((16,), j, jnp.int32),
                                             rows16[t], col])
                stage_v[j, d, pl.ds(t * 16, 16)] = (v * 8.0 + p) * m1[t]

    load_and_fire(0, 0)

    def body(s2, carry):
        s = s2 * 2
        load_and_fire(s + 1, 1)

        @pl.when(s2 > 0)
        def _():
            wait_out(s - 2, 0)
        compute(s, 0)
        fire_out(s, 0)

        @pl.when(s2 < _S // 2 - 1)
        def _():
            load_and_fire(s + 2, 0)

        @pl.when(s2 > 0)
        def _():
            wait_out(s - 1, 1)
        compute(s + 1, 1)
        fire_out(s + 1, 1)
        return carry

    lax.fori_loop(0, _S // 2, body, 0)
    wait_out(_S - 2, 0)
    wait_out(_S - 1, 1)


def kernel(inputs, token_table, pos_table):
    ids4 = (inputs.T.reshape(_S // 8, 8, _B // _C, _C)
            .transpose(0, 2, 1, 3))
    tok_half = token_table.reshape(2 * _V, _H)
    out5 = _embed(ids4, tok_half, pos_table)
    return (out5.transpose(2, 4, 0, 1, 3)
            .reshape(_B, _S, _D))


# speedup vs baseline: 2.4678x; 1.9962x over previous
import functools

import jax
import jax.numpy as jnp
from jax import lax
from jax.experimental import pallas as pl
from jax.experimental.pallas import tpu as pltpu
from jax.experimental.pallas import tpu_sc as plsc

_B = 4096
_S = 200
_D = 64
_V = 1000000

_NC = 2
_NS = 16
_NW = _NC * _NS
_C = _B // _NW
_H = _D // 2

_mesh = plsc.VectorSubcoreMesh(core_axis_name="c", subcore_axis_name="s")


@functools.partial(
    pl.kernel,
    out_type=jax.ShapeDtypeStruct((_S, _D // 8, _B // _C, 8, _C), jnp.float32),
    mesh=_mesh,
    compiler_params=pltpu.CompilerParams(
        needs_layout_passes=False, use_tc_tiling_on_sc=False),
    scratch_types=[
        pltpu.VMEM((_S // 8, 8, _C), jnp.int32),
        pltpu.VMEM((2, _C), jnp.int32),
        pltpu.VMEM((2, _C), jnp.int32),
        pltpu.VMEM((_C, _H), jnp.float32),
        pltpu.VMEM((_C, _H), jnp.float32),
        pltpu.VMEM((_C, _H), jnp.float32),
        pltpu.VMEM((_C, _H), jnp.float32),
        pltpu.VMEM((_D, _C), jnp.float32),
        pltpu.VMEM((_D, _C), jnp.float32),
        pltpu.VMEM((_S, _D), jnp.float32),
        pltpu.SemaphoreType.DMA((2,)),
        pltpu.SemaphoreType.DMA((2,)),
    ],
)
def _embed(ids_hbm, tok_hbm, pos_hbm, out_hbm,
           ids_v, eidx_v, oidx_v, even0_v, even1_v, odd0_v, odd1_v,
           stage0_v, stage1_v, pos_v, gsem, osem):
    wid = lax.axis_index("s") * _NC + lax.axis_index("c")
    pltpu.sync_copy(pos_hbm, pos_v)
    for k in range(_S // 8):
        pltpu.sync_copy(ids_hbm.at[k, wid], ids_v.at[k])
    lane = jnp.arange(16, dtype=jnp.int32)
    evens = (even0_v, even1_v)
    odds = (odd0_v, odd1_v)
    stages = (stage0_v, stage1_v)

    def fire_gathers(s, j):
        sh = lax.div(s, 8)
        sl = lax.rem(s, 8)

        @plsc.parallel_loop(0, _C // 16)
        def _expand(t):
            tv = ids_v[sh, sl, pl.ds(t * 16, 16)]
            tv4 = tv * 4
            eidx_v[j, pl.ds(t * 16, 16)] = tv4
            oidx_v[j, pl.ds(t * 16, 16)] = tv4 + 1

        pltpu.async_copy(tok_hbm.at[eidx_v.at[j]], evens[j], gsem.at[j])
        pltpu.async_copy(tok_hbm.at[oidx_v.at[j]], odds[j], gsem.at[j])

    def wait_gathers(j):
        pltpu.make_async_copy(tok_hbm.at[eidx_v.at[j]], evens[j],
                              gsem.at[j]).wait()
        pltpu.make_async_copy(tok_hbm.at[oidx_v.at[j]], odds[j],
                              gsem.at[j]).wait()

    def fire_out(s, j):
        for dh in range(_D // 8):
            pltpu.async_copy(stages[j].at[pl.ds(dh * 8, 8)],
                             out_hbm.at[s, dh, wid], osem.at[j])

    def wait_out(s, j):
        for dh in range(_D // 8):
            pltpu.make_async_copy(stages[j].at[pl.ds(dh * 8, 8)],
                                  out_hbm.at[s, dh, wid], osem.at[j]).wait()

    def compute(s, j):
        sh = lax.div(s, 8)
        sl = lax.rem(s, 8)
        wait_gathers(j)
        even_v, odd_v, stage_v = evens[j], odds[j], stages[j]
        m1 = []
        rows16 = []
        for t in range(_C // 16):
            tv = ids_v[sh, sl, pl.ds(t * 16, 16)]
            m1.append(jnp.where(tv != 0, 1.0, 0.0))
            rows16.append(lane + (t * 16))

        @functools.partial(plsc.parallel_loop, 0, _H, unroll=2)
        def _col(d):
            pe = plsc.load_gather(pos_v, [jnp.full((16,), s, jnp.int32),
                                          jnp.full((16,), d, jnp.int32)])
            po = plsc.load_gather(pos_v, [jnp.full((16,), s, jnp.int32),
                                          jnp.full((16,), d + _H, jnp.int32)])
            col = jnp.full((16,), d, jnp.int32)
            for t in range(_C // 16):
                ve = plsc.load_gather(even_v, [rows16[t], col])
                stage_v[d, pl.ds(t * 16, 16)] = (ve * 8.0 + pe) * m1[t]
                vo = plsc.load_gather(odd_v, [rows16[t], col])
                stage_v[d + _H, pl.ds(t * 16, 16)] = (vo * 8.0 + po) * m1[t]

    fire_gathers(0, 0)

    def body(s2, carry):
        s = s2 * 2
        fire_gathers(s + 1, 1)

        @pl.when(s2 > 0)
        def _():
            wait_out(s - 2, 0)
        compute(s, 0)
        fire_out(s, 0)

        @pl.when(s2 < _S // 2 - 1)
        def _():
            fire_gathers(s + 2, 0)

        @pl.when(s2 > 0)
        def _():
            wait_out(s - 1, 1)
        compute(s + 1, 1)
        fire_out(s + 1, 1)
        return carry

    lax.fori_loop(0, _S // 2, body, 0)
    wait_out(_S - 2, 0)
    wait_out(_S - 1, 1)


def kernel(inputs, token_table, pos_table):
    ids4 = (inputs.T.reshape(_S // 8, 8, _B // _C, _C)
            .transpose(0, 2, 1, 3))
    tok_pad = jnp.pad(token_table, ((0, 0), (0, _D)))
    tok_half = tok_pad.reshape(4 * _V, _H)
    out5 = _embed(ids4, tok_half, pos_table)
    return (out5.transpose(2, 4, 0, 1, 3)
            .reshape(_B, _S, _D))
